# baseline (speedup 1.0000x reference)
import numpy as np
import jax
import jax.numpy as jnp
from jax import lax
from jax.experimental import pallas as pl
from jax.experimental.pallas import tpu as pltpu

N_KERNELS = 10000
IN_CH = 3
BATCH = 64
T_LEN = 1024
KSIZES = [7, 9, 11]
DILS = [1, 2, 4, 8, 16]
MAXK = 11
PAD = (MAXK // 2) * max(DILS)
T_PAD = T_LEN + 2 * PAD

_rng = np.random.default_rng(0)
_ks = np.array(KSIZES)[_rng.integers(0, len(KSIZES), N_KERNELS)]
_dil = np.array(DILS)[_rng.integers(0, len(DILS), N_KERNELS)]

N_DIL = len(DILS)
SHIFTS = sorted({(p - MAXK // 2) * d for d in DILS for p in range(MAXK)})
N_SHIFT = len(SHIFTS)
K_ROWS = N_SHIFT * IN_CH + 1
K_PAD = 128
NK_PAD = 10240
N_TILES = NK_PAD // 256
M_CHUNK = 256

_shift = (MAXK - _ks) // 2
_shift_onehot = [(_shift == s).astype(np.float32) for s in range(3)]
_dgi = np.searchsorted(np.array(DILS), _dil)
_dil_onehot = (np.arange(N_DIL)[None, :] == _dgi[:, None]).astype(np.float32)
_sidx = {s: i for i, s in enumerate(SHIFTS)}
_P2 = np.zeros((N_DIL * IN_CH * MAXK, K_PAD), np.float32)
for _g, _d in enumerate(DILS):
    for _c in range(IN_CH):
        for _p in range(MAXK):
            _P2[_g * IN_CH * MAXK + _c * MAXK + _p,
                _c * N_SHIFT + _sidx[(_p - MAXK // 2) * _d]] = 1.0
_BIAS_ROW = np.zeros((K_PAD,), np.float32)
_BIAS_ROW[K_ROWS - 1] = 1.0
_ROWS = [(c, s) for c in range(IN_CH) for s in SHIFTS]


def _body(xp_ref, w_ref, mx_ref, pv_ref, xcs_ref, xt_ref):
    for r, (c, s) in enumerate(_ROWS):
        xcs_ref[r:r + 1, :] = xp_ref[c:c + 1, PAD + s: PAD + s + T_LEN]
    xcs_ref[K_ROWS - 1:K_ROWS, :] = jnp.ones((1, T_LEN), jnp.float32)
    xcs_ref[K_ROWS:, :] = jnp.zeros((K_PAD - K_ROWS, T_LEN), jnp.float32)
    xt_ref[...] = xcs_ref[...].T
    for j in range(N_TILES):
        w_tile = w_ref[j]
        mx8 = jnp.full((8, 256), -jnp.inf, jnp.float32)
        ng8 = jnp.zeros((8, 256), jnp.int32)
        for c in range(T_LEN // M_CHUNK):
            lhs = xt_ref[c * M_CHUNK:(c + 1) * M_CHUNK, :]
            out = lax.dot_general(
                lhs, w_tile, (((1,), (0,)), ((), ())),
                preferred_element_type=jnp.float32)
            o3 = out.reshape(M_CHUNK // 8, 8, 256)
            mx8 = jnp.maximum(mx8, jnp.max(o3, axis=0))
            neg = lax.shift_right_logical(
                lax.bitcast_convert_type(o3, jnp.uint32), np.uint32(31))
            ng8 = ng8 + jnp.sum(neg.astype(jnp.int32), axis=0)
        sl = slice(j * 256, (j + 1) * 256)
        mx_ref[:, sl] = jnp.max(mx8, axis=0, keepdims=True)
        cnt = jnp.sum(ng8, axis=0, keepdims=True)
        pv_ref[:, sl] = 1.0 - cnt.astype(jnp.float32) * (1.0 / T_LEN)


def _features(x, weights, biases):
    f32 = jnp.float32
    batch = x.shape[0]
    wt = weights.transpose(1, 2, 0)
    w11 = sum(jnp.asarray(m)[None, None, :] * jnp.roll(wt, s, axis=1)
              for s, m in enumerate(_shift_onehot))
    w_expt = jnp.asarray(_dil_onehot.T)[:, None, None, :] * w11[None]
    w_flatt = w_expt.reshape(N_DIL * IN_CH * MAXK, N_KERNELS)
    w_kmaj = jnp.dot(jnp.asarray(_P2.T), w_flatt)
    w_kmaj = w_kmaj + jnp.asarray(_BIAS_ROW)[:, None] * biases[None, :]
    w_kmaj = jnp.pad(w_kmaj, ((0, 0), (0, NK_PAD - N_KERNELS)))
    w3 = w_kmaj.reshape(K_PAD, N_TILES, 256).transpose(1, 0, 2)

    xpad = jnp.pad(x, ((0, 0), (0, 0), (PAD, PAD)))

    mx, pv = pl.pallas_call(
        _body,
        grid=(batch,),
        in_specs=[
            pl.BlockSpec((None, IN_CH, T_PAD), lambda b: (b, 0, 0)),
            pl.BlockSpec((N_TILES, K_PAD, 256), lambda b: (0, 0, 0)),
        ],
        out_specs=[
            pl.BlockSpec((None, 1, NK_PAD), lambda b: (b, 0, 0)),
            pl.BlockSpec((None, 1, NK_PAD), lambda b: (b, 0, 0)),
        ],
        out_shape=[jax.ShapeDtypeStruct((batch, 1, NK_PAD), f32)] * 2,
        scratch_shapes=[pltpu.VMEM((K_PAD, T_LEN), jnp.float32),
                        pltpu.VMEM((T_LEN, K_PAD), jnp.float32)],
        compiler_params=pltpu.CompilerParams(
            dimension_semantics=("parallel",)),
    )(xpad, w3)

    mx = mx[:, 0, :N_KERNELS]
    pv = pv[:, 0, :N_KERNELS]
    return jnp.stack([mx, pv], -1).reshape(batch, 2 * N_KERNELS)


def kernel(x, weights, biases):
    devs = jax.devices()
    n_dev = len(devs) if (devs and BATCH % max(len(devs), 1) == 0) else 1
    if n_dev <= 1:
        return _features(x, weights, biases)
    from jax.sharding import Mesh, PartitionSpec as P
    mesh = Mesh(np.array(devs), ("b",))
    f = jax.shard_map(
        _features, mesh=mesh,
        in_specs=(P("b"), P(), P()),
        out_specs=P("b"), check_vma=False)
    return f(x, weights, biases)


# speedup vs baseline: 1.4946x; 1.4946x over previous
import numpy as np
import jax
import jax.numpy as jnp
from jax import lax
from jax.experimental import pallas as pl
from jax.experimental.pallas import tpu as pltpu

N_KERNELS = 10000
IN_CH = 3
BATCH = 64
T_LEN = 1024
KSIZES = [7, 9, 11]
DILS = [1, 2, 4, 8, 16]
MAXK = 11
PAD = (MAXK // 2) * max(DILS)
T_PAD = T_LEN + 2 * PAD

_rng = np.random.default_rng(0)
_ks = np.array(KSIZES)[_rng.integers(0, len(KSIZES), N_KERNELS)]
_dil = np.array(DILS)[_rng.integers(0, len(DILS), N_KERNELS)]

N_DIL = len(DILS)
SHIFTS = sorted({(p - MAXK // 2) * d for d in DILS for p in range(MAXK)})
N_SHIFT = len(SHIFTS)
K_ROWS = N_SHIFT * IN_CH + 1
K_PAD = 128
NK_PAD = 10240
N_TILES = NK_PAD // 256
M_CHUNK = 256

_shift = (MAXK - _ks) // 2
_shift_onehot = [(_shift == s).astype(np.float32) for s in range(3)]
_dgi = np.searchsorted(np.array(DILS), _dil)
_dil_onehot = (np.arange(N_DIL)[None, :] == _dgi[:, None]).astype(np.float32)
_sidx = {s: i for i, s in enumerate(SHIFTS)}
_P2 = np.zeros((N_DIL * IN_CH * MAXK, K_PAD), np.float32)
for _g, _d in enumerate(DILS):
    for _c in range(IN_CH):
        for _p in range(MAXK):
            _P2[_g * IN_CH * MAXK + _c * MAXK + _p,
                _c * N_SHIFT + _sidx[(_p - MAXK // 2) * _d]] = 1.0
_BIAS_ROW = np.zeros((K_PAD,), np.float32)
_BIAS_ROW[K_ROWS - 1] = 1.0
_ROWS = [(c, s) for c in range(IN_CH) for s in SHIFTS]


def _body(xp_ref, w_ref, mx_ref, pv_ref, xcs_ref, xt_ref):
    for r, (c, s) in enumerate(_ROWS):
        xcs_ref[r:r + 1, :] = xp_ref[c:c + 1, PAD + s: PAD + s + T_LEN]
    xcs_ref[K_ROWS - 1:K_ROWS, :] = jnp.ones((1, T_LEN), jnp.float32)
    xcs_ref[K_ROWS:, :] = jnp.zeros((K_PAD - K_ROWS, T_LEN), jnp.float32)
    xt_ref[...] = xcs_ref[...].T
    for j in range(N_TILES):
        w_tile = w_ref[j]
        mx8 = jnp.full((8, 256), -jnp.inf, jnp.float32)
        ng8 = jnp.zeros((8, 256), jnp.int32)
        for c in range(T_LEN // M_CHUNK):
            lhs = xt_ref[c * M_CHUNK:(c + 1) * M_CHUNK, :]
            out = lax.dot_general(
                lhs, w_tile, (((1,), (0,)), ((), ())),
                preferred_element_type=jnp.float32)
            o3 = out.reshape(M_CHUNK // 8, 8, 256)
            mx8 = jnp.maximum(mx8, jnp.max(o3, axis=0))
            neg = lax.shift_right_logical(
                lax.bitcast_convert_type(o3, jnp.uint32), np.uint32(31))
            ng8 = ng8 + jnp.sum(neg.astype(jnp.int32), axis=0)
        sl = slice(j * 256, (j + 1) * 256)
        mx_ref[:, sl] = jnp.max(mx8, axis=0, keepdims=True)
        cnt = jnp.sum(ng8, axis=0, keepdims=True)
        pv_ref[:, sl] = 1.0 - cnt.astype(jnp.float32) * (1.0 / T_LEN)


def kernel(x, weights, biases):
    f32 = jnp.float32
    wt = weights.transpose(1, 2, 0)
    w11 = sum(jnp.asarray(m)[None, None, :] * jnp.roll(wt, s, axis=1)
              for s, m in enumerate(_shift_onehot))
    w_expt = jnp.asarray(_dil_onehot.T)[:, None, None, :] * w11[None]
    w_flatt = w_expt.reshape(N_DIL * IN_CH * MAXK, N_KERNELS)
    w_kmaj = jnp.dot(jnp.asarray(_P2.T), w_flatt)
    w_kmaj = w_kmaj + jnp.asarray(_BIAS_ROW)[:, None] * biases[None, :]
    w_kmaj = jnp.pad(w_kmaj, ((0, 0), (0, NK_PAD - N_KERNELS)))
    w3 = w_kmaj.reshape(K_PAD, N_TILES, 256).transpose(1, 0, 2)

    xpad = jnp.pad(x, ((0, 0), (0, 0), (PAD, PAD)))

    mx, pv = pl.pallas_call(
        _body,
        grid=(BATCH,),
        in_specs=[
            pl.BlockSpec((None, IN_CH, T_PAD), lambda b: (b, 0, 0)),
            pl.BlockSpec((N_TILES, K_PAD, 256), lambda b: (0, 0, 0)),
        ],
        out_specs=[
            pl.BlockSpec((None, 1, NK_PAD), lambda b: (b, 0, 0)),
            pl.BlockSpec((None, 1, NK_PAD), lambda b: (b, 0, 0)),
        ],
        out_shape=[jax.ShapeDtypeStruct((BATCH, 1, NK_PAD), f32)] * 2,
        scratch_shapes=[pltpu.VMEM((K_PAD, T_LEN), jnp.float32),
                        pltpu.VMEM((T_LEN, K_PAD), jnp.float32)],
        compiler_params=pltpu.CompilerParams(
            dimension_semantics=("parallel",)),
    )(xpad, w3)

    mx = mx[:, 0, :N_KERNELS]
    pv = pv[:, 0, :N_KERNELS]
    return jnp.stack([mx, pv], -1).reshape(BATCH, 2 * N_KERNELS)
